# Initial kernel scaffold; baseline (speedup 1.0000x reference)
import jax
import jax.numpy as jnp
from jax.experimental import pallas as pl


def kernel(x, edge_index, batch, emb, W1, a_src1, a_dst1, b1, W2, a_src2, a_dst2, b2, W3, a_src3, a_dst3, b3, W4, a_src4, a_dst4, b4, W5, a_src5, a_dst5, b5, Wf, bf):
    raise NotImplementedError("write your pallas kernel here")


# speedup vs baseline: 40.2438x; 40.2438x over previous
import functools

import jax
import jax.numpy as jnp
from jax import lax
from jax.experimental import pallas as pl
from jax.experimental.pallas import tpu as pltpu
from jax.experimental.pallas import tpu_sc as plsc

N = 10000
E = 320000
G = 256
D = 128
H = 8
C = 16
C2 = 16

R = 256
N_PAD = 10240
NBLK = N_PAD // R

NC = 2
NS = 16
NW = NC * NS
CH = 120
NITER = 86
EPT = CH * NITER
E2P = NW * EPT
ROWS_PT = N_PAD // NS

NEG = -1e30

_f32 = jnp.float32


def _dense_tail(i, hin, w_ref, afs_ref, afd_ref, h_ref, as_ref, ad_ref):
    h = jnp.dot(hin, w_ref[...], preferred_element_type=_f32)
    d_io = lax.broadcasted_iota(jnp.int32, (D, C2), 0)
    j_io = lax.broadcasted_iota(jnp.int32, (D, C2), 1)
    sel = ((d_io // C) == j_io).astype(_f32)
    a_s = jnp.dot(h * afs_ref[...], sel, preferred_element_type=_f32)
    a_d = jnp.dot(h * afd_ref[...], sel, preferred_element_type=_f32)
    row = i * R + lax.broadcasted_iota(jnp.int32, (R, 1), 0)
    valid = row < N
    as_ref[...] = jnp.where(valid, a_s, NEG)
    ad_ref[...] = jnp.where(valid, a_d, NEG)
    h_ref[...] = h


def _embed_dense_body(x_ref, emb_ref, w_ref, afs_ref, afd_ref,
                      h_ref, as_ref, ad_ref):
    i = pl.program_id(0)
    emb = emb_ref[...]
    nrm = jnp.sqrt(jnp.sum(emb * emb, axis=1, keepdims=True))
    scale = jnp.minimum(1.0, 1.0 / jnp.maximum(nrm, 1e-12))
    embn = emb * scale
    col = lax.broadcasted_iota(jnp.int32, (1, 72), 1)
    oh = (x_ref[...] == col).astype(_f32)
    hin = jnp.dot(oh, embn, preferred_element_type=_f32)
    _dense_tail(i, hin, w_ref, afs_ref, afd_ref, h_ref, as_ref, ad_ref)


def _dense_body(a0_ref, a1_ref, b_ref, w_ref, afs_ref, afd_ref,
                h_ref, as_ref, ad_ref):
    i = pl.program_id(0)
    hin = jnp.maximum(a0_ref[...] + a1_ref[...] + b_ref[...], 0.0)
    _dense_tail(i, hin, w_ref, afs_ref, afd_ref, h_ref, as_ref, ad_ref)


def _inv_body(d0_ref, d1_ref, o_ref):
    o_ref[...] = 1.0 / (d0_ref[...] + d1_ref[...] + 1e-16)


def _pool_body(batch_ref, a0_ref, a1_ref, b_ref, wf_ref, bf_ref,
               y_ref, acc_ref):
    nj = pl.program_id(1)

    @pl.when(nj == 0)
    def _():
        acc_ref[...] = jnp.full((8, D), NEG, _f32)

    hb = jnp.maximum(a0_ref[...] + a1_ref[...] + b_ref[...], 0.0)
    gids = pl.program_id(0) * 8 + lax.broadcasted_iota(jnp.int32, (1, 8), 1)
    m = batch_ref[...] == gids
    rows = [jnp.max(jnp.where(m[:, k:k + 1], hb, NEG), axis=0, keepdims=True)
            for k in range(8)]
    acc_ref[...] = jnp.maximum(acc_ref[...], jnp.concatenate(rows, axis=0))
    pooled = acc_ref[...]
    pooled = jnp.where(pooled > -1e29, pooled, 0.0)
    y_ref[...] = (jnp.sum(pooled * wf_ref[...], axis=1, keepdims=True)
                  + bf_ref[:, :1])


_dense1_call = pl.pallas_call(
    _embed_dense_body,
    grid=(NBLK,),
    in_specs=[
        pl.BlockSpec((R, 1), lambda i: (i, 0)),
        pl.BlockSpec((72, D), lambda i: (0, 0)),
        pl.BlockSpec((D, D), lambda i: (0, 0)),
        pl.BlockSpec((1, D), lambda i: (0, 0)),
        pl.BlockSpec((1, D), lambda i: (0, 0)),
    ],
    out_specs=[
        pl.BlockSpec((R, D), lambda i: (i, 0)),
        pl.BlockSpec((R, C2), lambda i: (i, 0)),
        pl.BlockSpec((R, C2), lambda i: (i, 0)),
    ],
    out_shape=[
        jax.ShapeDtypeStruct((N_PAD, D), _f32),
        jax.ShapeDtypeStruct((N_PAD, C2), _f32),
        jax.ShapeDtypeStruct((N_PAD, C2), _f32),
    ],
)

_dense_call = pl.pallas_call(
    _dense_body,
    grid=(NBLK,),
    in_specs=[
        pl.BlockSpec((R, D), lambda i: (i, 0)),
        pl.BlockSpec((R, D), lambda i: (i, 0)),
        pl.BlockSpec((1, D), lambda i: (0, 0)),
        pl.BlockSpec((D, D), lambda i: (0, 0)),
        pl.BlockSpec((1, D), lambda i: (0, 0)),
        pl.BlockSpec((1, D), lambda i: (0, 0)),
    ],
    out_specs=[
        pl.BlockSpec((R, D), lambda i: (i, 0)),
        pl.BlockSpec((R, C2), lambda i: (i, 0)),
        pl.BlockSpec((R, C2), lambda i: (i, 0)),
    ],
    out_shape=[
        jax.ShapeDtypeStruct((N_PAD, D), _f32),
        jax.ShapeDtypeStruct((N_PAD, C2), _f32),
        jax.ShapeDtypeStruct((N_PAD, C2), _f32),
    ],
)

_inv_call = pl.pallas_call(
    _inv_body,
    grid=(NBLK,),
    in_specs=[
        pl.BlockSpec((R, C2), lambda i: (i, 0)),
        pl.BlockSpec((R, C2), lambda i: (i, 0)),
    ],
    out_specs=pl.BlockSpec((R, C2), lambda i: (i, 0)),
    out_shape=jax.ShapeDtypeStruct((N_PAD, C2), _f32),
)

_pool_call = pl.pallas_call(
    _pool_body,
    grid=(G // 8, NBLK),
    in_specs=[
        pl.BlockSpec((R, 1), lambda g, j: (j, 0)),
        pl.BlockSpec((R, D), lambda g, j: (j, 0)),
        pl.BlockSpec((R, D), lambda g, j: (j, 0)),
        pl.BlockSpec((1, D), lambda g, j: (0, 0)),
        pl.BlockSpec((1, D), lambda g, j: (0, 0)),
        pl.BlockSpec((1, D), lambda g, j: (0, 0)),
    ],
    out_specs=pl.BlockSpec((8, 1), lambda g, j: (g, 0)),
    out_shape=jax.ShapeDtypeStruct((G, 1), _f32),
    scratch_shapes=[pltpu.VMEM((8, D), _f32)],
)


_mesh = plsc.VectorSubcoreMesh(
    core_axis_name="c", subcore_axis_name="s", num_cores=NC, num_subcores=NS)


def _sc_pass1(as_hbm, ad_hbm, src_hbm, dst_hbm, z16_hbm,
              ex_hbm, den_hbm,
              idx_s, idx_d, sbuf, dbuf, ebuf, den_sh, sem):
    cid = lax.axis_index("c")
    sid = lax.axis_index("s")
    wid = cid * NS + sid
    r0 = sid * ROWS_PT
    pltpu.sync_copy(z16_hbm.at[pl.ds(r0, ROWS_PT)],
                    den_sh.at[pl.ds(r0, ROWS_PT)])
    plsc.subcore_barrier()

    def step(i, carry):
        base = wid * EPT + i * CH
        pltpu.sync_copy(src_hbm.at[pl.ds(base, CH)], idx_s.at[0])
        pltpu.sync_copy(dst_hbm.at[pl.ds(base, CH)], idx_d.at[0])
        pltpu.async_copy(as_hbm.at[idx_s.at[0]], sbuf, sem).wait()
        pltpu.async_copy(ad_hbm.at[idx_d.at[0]], dbuf, sem).wait()

        def inner(e, c2):
            t = sbuf[e, :] + dbuf[e, :]
            ebuf[e, :] = jnp.exp(jnp.where(t > 0, t, 0.2 * t))
            return c2

        lax.fori_loop(0, CH, inner, 0)
        pltpu.sync_copy(ebuf, ex_hbm.at[pl.ds(base, CH)])
        pltpu.sync_copy(ebuf, den_sh.at[idx_d.at[0]], add=True)
        return carry

    lax.fori_loop(0, NITER, step, 0)
    plsc.subcore_barrier()
    pltpu.sync_copy(den_sh.at[pl.ds(r0, ROWS_PT)],
                    den_hbm.at[cid, pl.ds(r0, ROWS_PT)])


def _sc_pass2(h_hbm, ex_hbm, inv_hbm, src_hbm, dst_hbm, z128_hbm,
              out_hbm,
              idx_s, idx_d, hbuf, ebuf, ibuf, out_sh, sem):
    cid = lax.axis_index("c")
    sid = lax.axis_index("s")
    wid = cid * NS + sid
    r0 = sid * ROWS_PT
    pltpu.sync_copy(z128_hbm.at[pl.ds(r0, ROWS_PT)],
                    out_sh.at[pl.ds(r0, ROWS_PT)])
    plsc.subcore_barrier()

    def step(i, carry):
        base = wid * EPT + i * CH
        pltpu.sync_copy(src_hbm.at[pl.ds(base, CH)], idx_s.at[0])
        pltpu.sync_copy(dst_hbm.at[pl.ds(base, CH)], idx_d.at[0])
        pltpu.async_copy(h_hbm.at[idx_s.at[0]], hbuf, sem).wait()
        pltpu.sync_copy(ex_hbm.at[pl.ds(base, CH)], ebuf)
        pltpu.async_copy(inv_hbm.at[idx_d.at[0]], ibuf, sem).wait()

        def inner(e, c2):
            ev = ebuf[e, :] * ibuf[e, :]
            for k in range(H):
                coef = ev[k]
                hbuf[e, pl.ds(k * C, C)] = hbuf[e, pl.ds(k * C, C)] * coef
            return c2

        lax.fori_loop(0, CH, inner, 0)
        pltpu.sync_copy(hbuf, out_sh.at[idx_d.at[0]], add=True)
        return carry

    lax.fori_loop(0, NITER, step, 0)
    plsc.subcore_barrier()
    pltpu.sync_copy(out_sh.at[pl.ds(r0, ROWS_PT)],
                    out_hbm.at[cid, pl.ds(r0, ROWS_PT)])


_sc_params = pltpu.CompilerParams(use_tc_tiling_on_sc=False)

_pass1_call = pl.kernel(
    _sc_pass1,
    out_type=(
        jax.ShapeDtypeStruct((E2P, C2), _f32),
        jax.ShapeDtypeStruct((NC, N_PAD, C2), _f32),
    ),
    mesh=_mesh,
    compiler_params=_sc_params,
    scratch_types=[
        pltpu.VMEM((1, CH), jnp.int32),
        pltpu.VMEM((1, CH), jnp.int32),
        pltpu.VMEM((CH, C2), _f32),
        pltpu.VMEM((CH, C2), _f32),
        pltpu.VMEM((CH, C2), _f32),
        pltpu.VMEM_SHARED((N_PAD, C2), _f32),
        pltpu.SemaphoreType.DMA,
    ],
)

_pass2_call = pl.kernel(
    _sc_pass2,
    out_type=jax.ShapeDtypeStruct((NC, N_PAD, D), _f32),
    mesh=_mesh,
    compiler_params=_sc_params,
    scratch_types=[
        pltpu.VMEM((1, CH), jnp.int32),
        pltpu.VMEM((1, CH), jnp.int32),
        pltpu.VMEM((CH, D), _f32),
        pltpu.VMEM((CH, C2), _f32),
        pltpu.VMEM((CH, C2), _f32),
        pltpu.VMEM_SHARED((N_PAD, D), _f32),
        pltpu.SemaphoreType.DMA,
    ],
)


def kernel(x, edge_index, batch, emb,
           W1, a_src1, a_dst1, b1, W2, a_src2, a_dst2, b2,
           W3, a_src3, a_dst3, b3, W4, a_src4, a_dst4, b4,
           W5, a_src5, a_dst5, b5, Wf, bf):
    x2 = jnp.zeros((N_PAD, 1), jnp.int32).at[:N, 0].set(x.astype(jnp.int32))
    embp = jnp.zeros((72, D), _f32).at[:65].set(emb)
    npad = E2P - E - N
    src = jnp.concatenate([
        edge_index[0].astype(jnp.int32),
        jnp.arange(N, dtype=jnp.int32),
        jnp.full((npad,), N, jnp.int32),
    ])
    dst = jnp.concatenate([
        edge_index[1].astype(jnp.int32),
        jnp.arange(N, dtype=jnp.int32),
        jnp.full((npad,), N, jnp.int32),
    ])
    batch2 = jnp.full((N_PAD, 1), G + 8, jnp.int32)
    batch2 = batch2.at[:N, 0].set(batch.astype(jnp.int32))
    z16 = jnp.zeros((N_PAD, C2), _f32)
    z128 = jnp.zeros((N_PAD, D), _f32)

    layers = [
        (W1, a_src1, a_dst1, b1),
        (W2, a_src2, a_dst2, b2),
        (W3, a_src3, a_dst3, b3),
        (W4, a_src4, a_dst4, b4),
        (W5, a_src5, a_dst5, b5),
    ]

    h, a_s, a_d = _dense1_call(
        x2, embp, W1, a_src1.reshape(1, D), a_dst1.reshape(1, D))

    for li in range(5):
        ex, den = _pass1_call(a_s, a_d, src, dst, z16)
        inv = _inv_call(den[0], den[1])
        outp = _pass2_call(h, ex, inv, src, dst, z128)
        if li < 4:
            w, asr, adr, b = layers[li + 1]
            h, a_s, a_d = _dense_call(
                outp[0], outp[1], b.reshape(1, D), w,
                asr.reshape(1, D), adr.reshape(1, D))
        else:
            y = _pool_call(
                batch2, outp[0], outp[1], b5.reshape(1, D),
                Wf.reshape(1, D),
                jnp.broadcast_to(bf.reshape(1, 1), (1, D)))
    return y.reshape(-1)


# speedup vs baseline: 51.3591x; 1.2762x over previous
import functools

import jax
import jax.numpy as jnp
from jax import lax
from jax.experimental import pallas as pl
from jax.experimental.pallas import tpu as pltpu
from jax.experimental.pallas import tpu_sc as plsc

N = 10000
E = 320000
G = 256
D = 128
H = 8
C = 16
C2 = 16

R = 256
N_PAD = 10240
NBLK = N_PAD // R

NC = 2
NS = 16
NW = NC * NS
CH = 120
NITER = 86
EPT = CH * NITER
E2P = NW * EPT
ROWS_PT = N_PAD // NS

NEG = -1e30

_f32 = jnp.float32


def _dense_tail(i, hin, w_ref, afs_ref, afd_ref, h_ref, as_ref, ad_ref):
    h = jnp.dot(hin, w_ref[...], preferred_element_type=_f32)
    d_io = lax.broadcasted_iota(jnp.int32, (D, C2), 0)
    j_io = lax.broadcasted_iota(jnp.int32, (D, C2), 1)
    sel = ((d_io // C) == j_io).astype(_f32)
    a_s = jnp.dot(h * afs_ref[...], sel, preferred_element_type=_f32)
    a_d = jnp.dot(h * afd_ref[...], sel, preferred_element_type=_f32)
    row = i * R + lax.broadcasted_iota(jnp.int32, (R, 1), 0)
    valid = row < N
    as_ref[...] = jnp.where(valid, a_s, NEG)
    ad_ref[...] = jnp.where(valid, a_d, NEG)
    h_ref[...] = h


def _embed_dense_body(x_ref, emb_ref, w_ref, afs_ref, afd_ref,
                      h_ref, as_ref, ad_ref):
    i = pl.program_id(0)
    emb = emb_ref[...]
    nrm = jnp.sqrt(jnp.sum(emb * emb, axis=1, keepdims=True))
    scale = jnp.minimum(1.0, 1.0 / jnp.maximum(nrm, 1e-12))
    embn = emb * scale
    col = lax.broadcasted_iota(jnp.int32, (1, 72), 1)
    oh = (x_ref[...] == col).astype(_f32)
    hin = jnp.dot(oh, embn, preferred_element_type=_f32)
    _dense_tail(i, hin, w_ref, afs_ref, afd_ref, h_ref, as_ref, ad_ref)


def _dense_body(a0_ref, a1_ref, b_ref, w_ref, afs_ref, afd_ref,
                h_ref, as_ref, ad_ref):
    i = pl.program_id(0)
    hin = jnp.maximum(a0_ref[...] + a1_ref[...] + b_ref[...], 0.0)
    _dense_tail(i, hin, w_ref, afs_ref, afd_ref, h_ref, as_ref, ad_ref)


def _inv_body(d0_ref, d1_ref, o_ref):
    o_ref[...] = 1.0 / (d0_ref[...] + d1_ref[...] + 1e-16)


def _pool_body(batch_ref, a0_ref, a1_ref, b_ref, wf_ref, bf_ref,
               y_ref, acc_ref):
    nj = pl.program_id(1)

    @pl.when(nj == 0)
    def _():
        acc_ref[...] = jnp.full((8, D), NEG, _f32)

    hb = jnp.maximum(a0_ref[...] + a1_ref[...] + b_ref[...], 0.0)
    gids = pl.program_id(0) * 8 + lax.broadcasted_iota(jnp.int32, (1, 8), 1)
    m = batch_ref[...] == gids
    rows = [jnp.max(jnp.where(m[:, k:k + 1], hb, NEG), axis=0, keepdims=True)
            for k in range(8)]
    acc_ref[...] = jnp.maximum(acc_ref[...], jnp.concatenate(rows, axis=0))
    pooled = acc_ref[...]
    pooled = jnp.where(pooled > -1e29, pooled, 0.0)
    y_ref[...] = (jnp.sum(pooled * wf_ref[...], axis=1, keepdims=True)
                  + bf_ref[:, :1])


_dense1_call = pl.pallas_call(
    _embed_dense_body,
    grid=(NBLK,),
    in_specs=[
        pl.BlockSpec((R, 1), lambda i: (i, 0)),
        pl.BlockSpec((72, D), lambda i: (0, 0)),
        pl.BlockSpec((D, D), lambda i: (0, 0)),
        pl.BlockSpec((1, D), lambda i: (0, 0)),
        pl.BlockSpec((1, D), lambda i: (0, 0)),
    ],
    out_specs=[
        pl.BlockSpec((R, D), lambda i: (i, 0)),
        pl.BlockSpec((R, C2), lambda i: (i, 0)),
        pl.BlockSpec((R, C2), lambda i: (i, 0)),
    ],
    out_shape=[
        jax.ShapeDtypeStruct((N_PAD, D), _f32),
        jax.ShapeDtypeStruct((N_PAD, C2), _f32),
        jax.ShapeDtypeStruct((N_PAD, C2), _f32),
    ],
)

_dense_call = pl.pallas_call(
    _dense_body,
    grid=(NBLK,),
    in_specs=[
        pl.BlockSpec((R, D), lambda i: (i, 0)),
        pl.BlockSpec((R, D), lambda i: (i, 0)),
        pl.BlockSpec((1, D), lambda i: (0, 0)),
        pl.BlockSpec((D, D), lambda i: (0, 0)),
        pl.BlockSpec((1, D), lambda i: (0, 0)),
        pl.BlockSpec((1, D), lambda i: (0, 0)),
    ],
    out_specs=[
        pl.BlockSpec((R, D), lambda i: (i, 0)),
        pl.BlockSpec((R, C2), lambda i: (i, 0)),
        pl.BlockSpec((R, C2), lambda i: (i, 0)),
    ],
    out_shape=[
        jax.ShapeDtypeStruct((N_PAD, D), _f32),
        jax.ShapeDtypeStruct((N_PAD, C2), _f32),
        jax.ShapeDtypeStruct((N_PAD, C2), _f32),
    ],
)

_inv_call = pl.pallas_call(
    _inv_body,
    grid=(NBLK,),
    in_specs=[
        pl.BlockSpec((R, C2), lambda i: (i, 0)),
        pl.BlockSpec((R, C2), lambda i: (i, 0)),
    ],
    out_specs=pl.BlockSpec((R, C2), lambda i: (i, 0)),
    out_shape=jax.ShapeDtypeStruct((N_PAD, C2), _f32),
)

_pool_call = pl.pallas_call(
    _pool_body,
    grid=(G // 8, NBLK),
    in_specs=[
        pl.BlockSpec((R, 1), lambda g, j: (j, 0)),
        pl.BlockSpec((R, D), lambda g, j: (j, 0)),
        pl.BlockSpec((R, D), lambda g, j: (j, 0)),
        pl.BlockSpec((1, D), lambda g, j: (0, 0)),
        pl.BlockSpec((1, D), lambda g, j: (0, 0)),
        pl.BlockSpec((1, D), lambda g, j: (0, 0)),
    ],
    out_specs=pl.BlockSpec((8, 1), lambda g, j: (g, 0)),
    out_shape=jax.ShapeDtypeStruct((G, 1), _f32),
    scratch_shapes=[pltpu.VMEM((8, D), _f32)],
)


_mesh = plsc.VectorSubcoreMesh(
    core_axis_name="c", subcore_axis_name="s", num_cores=NC, num_subcores=NS)


def _sc_pass1(as_hbm, ad_hbm, src_hbm, dst_hbm, z16_hbm,
              ex_hbm, den_hbm,
              idx_s, idx_d, sbuf, dbuf, ebuf, den_sh,
              semi, semg, semo0, semo1):
    cid = lax.axis_index("c")
    sid = lax.axis_index("s")
    wid = cid * NS + sid
    r0 = sid * ROWS_PT
    pltpu.sync_copy(z16_hbm.at[pl.ds(r0, ROWS_PT)],
                    den_sh.at[pl.ds(r0, ROWS_PT)])
    plsc.subcore_barrier()

    def chunk(i, p, semo):
        base = wid * EPT + i * CH
        i1 = pltpu.async_copy(src_hbm.at[wid, i], idx_s.at[0], semi)
        i2 = pltpu.async_copy(dst_hbm.at[wid, i], idx_d.at[0], semi)

        @pl.when(i >= 2)
        def _():
            pltpu.make_async_copy(
                ebuf.at[pl.ds(p, CH)], ex_hbm.at[pl.ds(base, CH)],
                semo).wait()

        i1.wait()
        i2.wait()
        g1 = pltpu.async_copy(as_hbm.at[idx_s.at[0]], sbuf, semg)
        g2 = pltpu.async_copy(ad_hbm.at[idx_d.at[0]], dbuf, semg)
        g1.wait()
        g2.wait()

        def inner(e, c2):
            t = sbuf[e, :] + dbuf[e, :]
            ebuf[p + e, :] = jnp.exp(jnp.where(t > 0, t, 0.2 * t))
            return c2

        lax.fori_loop(0, CH, inner, 0)
        pltpu.async_copy(ebuf.at[pl.ds(p, CH)],
                         ex_hbm.at[pl.ds(base, CH)], semo)
        pltpu.sync_copy(ebuf.at[pl.ds(p, CH)],
                        den_sh.at[idx_d.at[0]], add=True)

    def step(j, carry):
        chunk(2 * j, 0, semo0)
        chunk(2 * j + 1, CH, semo1)
        return carry

    lax.fori_loop(0, NITER // 2, step, 0)
    for i, p, semo in ((NITER - 2, 0, semo0), (NITER - 1, CH, semo1)):
        base = wid * EPT + i * CH
        pltpu.make_async_copy(
            ebuf.at[pl.ds(p, CH)], ex_hbm.at[pl.ds(base, CH)], semo).wait()
    plsc.subcore_barrier()
    pltpu.sync_copy(den_sh.at[pl.ds(r0, ROWS_PT)],
                    den_hbm.at[cid, pl.ds(r0, ROWS_PT)])


def _sc_pass2(h_hbm, ex_hbm, inv_hbm, src_hbm, dst_hbm, z128_hbm,
              out_hbm,
              idx_s, idx_d, hbuf, ebuf, ibuf, out_sh, semi, semg):
    cid = lax.axis_index("c")
    sid = lax.axis_index("s")
    wid = cid * NS + sid
    r0 = sid * ROWS_PT
    pltpu.sync_copy(z128_hbm.at[pl.ds(r0, ROWS_PT)],
                    out_sh.at[pl.ds(r0, ROWS_PT)])
    plsc.subcore_barrier()

    def step(i, carry):
        base = wid * EPT + i * CH
        i1 = pltpu.async_copy(src_hbm.at[wid, i], idx_s.at[0], semi)
        i2 = pltpu.async_copy(dst_hbm.at[wid, i], idx_d.at[0], semi)
        i1.wait()
        i2.wait()
        g1 = pltpu.async_copy(h_hbm.at[idx_s.at[0]], hbuf, semg)
        g2 = pltpu.async_copy(ex_hbm.at[pl.ds(base, CH)], ebuf, semg)
        g3 = pltpu.async_copy(inv_hbm.at[idx_d.at[0]], ibuf, semg)
        g1.wait()
        g2.wait()
        g3.wait()

        def inner(e, c2):
            ev = ebuf[e, :] * ibuf[e, :]
            for k in range(H):
                coef = ev[k]
                hbuf[e, pl.ds(k * C, C)] = hbuf[e, pl.ds(k * C, C)] * coef
            return c2

        lax.fori_loop(0, CH, inner, 0)
        pltpu.sync_copy(hbuf, out_sh.at[idx_d.at[0]], add=True)
        return carry

    lax.fori_loop(0, NITER, step, 0)
    plsc.subcore_barrier()
    pltpu.sync_copy(out_sh.at[pl.ds(r0, ROWS_PT)],
                    out_hbm.at[cid, pl.ds(r0, ROWS_PT)])


_sc_params = pltpu.CompilerParams(use_tc_tiling_on_sc=False)

_pass1_call = pl.kernel(
    _sc_pass1,
    out_type=(
        jax.ShapeDtypeStruct((E2P, C2), _f32),
        jax.ShapeDtypeStruct((NC, N_PAD, C2), _f32),
    ),
    mesh=_mesh,
    compiler_params=_sc_params,
    scratch_types=[
        pltpu.VMEM((1, CH), jnp.int32),
        pltpu.VMEM((1, CH), jnp.int32),
        pltpu.VMEM((CH, C2), _f32),
        pltpu.VMEM((CH, C2), _f32),
        pltpu.VMEM((2 * CH, C2), _f32),
        pltpu.VMEM_SHARED((N_PAD, C2), _f32),
        pltpu.SemaphoreType.DMA,
        pltpu.SemaphoreType.DMA,
        pltpu.SemaphoreType.DMA,
        pltpu.SemaphoreType.DMA,
    ],
)

_pass2_call = pl.kernel(
    _sc_pass2,
    out_type=jax.ShapeDtypeStruct((NC, N_PAD, D), _f32),
    mesh=_mesh,
    compiler_params=_sc_params,
    scratch_types=[
        pltpu.VMEM((1, CH), jnp.int32),
        pltpu.VMEM((1, CH), jnp.int32),
        pltpu.VMEM((CH, D), _f32),
        pltpu.VMEM((CH, C2), _f32),
        pltpu.VMEM((CH, C2), _f32),
        pltpu.VMEM_SHARED((N_PAD, D), _f32),
        pltpu.SemaphoreType.DMA,
        pltpu.SemaphoreType.DMA,
    ],
)


def kernel(x, edge_index, batch, emb,
           W1, a_src1, a_dst1, b1, W2, a_src2, a_dst2, b2,
           W3, a_src3, a_dst3, b3, W4, a_src4, a_dst4, b4,
           W5, a_src5, a_dst5, b5, Wf, bf):
    x2 = jnp.zeros((N_PAD, 1), jnp.int32).at[:N, 0].set(x.astype(jnp.int32))
    embp = jnp.zeros((72, D), _f32).at[:65].set(emb)
    npad = E2P - E - N
    src = jnp.concatenate([
        edge_index[0].astype(jnp.int32),
        jnp.arange(N, dtype=jnp.int32),
        jnp.full((npad,), N, jnp.int32),
    ])
    dst = jnp.concatenate([
        edge_index[1].astype(jnp.int32),
        jnp.arange(N, dtype=jnp.int32),
        jnp.full((npad,), N, jnp.int32),
    ])
    src3 = src.reshape(NW, NITER, CH)
    dst3 = dst.reshape(NW, NITER, CH)
    batch2 = jnp.full((N_PAD, 1), G + 8, jnp.int32)
    batch2 = batch2.at[:N, 0].set(batch.astype(jnp.int32))
    z16 = jnp.zeros((N_PAD, C2), _f32)
    z128 = jnp.zeros((N_PAD, D), _f32)

    layers = [
        (W1, a_src1, a_dst1, b1),
        (W2, a_src2, a_dst2, b2),
        (W3, a_src3, a_dst3, b3),
        (W4, a_src4, a_dst4, b4),
        (W5, a_src5, a_dst5, b5),
    ]

    h, a_s, a_d = _dense1_call(
        x2, embp, W1, a_src1.reshape(1, D), a_dst1.reshape(1, D))

    for li in range(5):
        ex, den = _pass1_call(a_s, a_d, src3, dst3, z16)
        inv = _inv_call(den[0], den[1])
        outp = _pass2_call(h, ex, inv, src3, dst3, z128)
        if li < 4:
            w, asr, adr, b = layers[li + 1]
            h, a_s, a_d = _dense_call(
                outp[0], outp[1], b.reshape(1, D), w,
                asr.reshape(1, D), adr.reshape(1, D))
        else:
            y = _pool_call(
                batch2, outp[0], outp[1], b5.reshape(1, D),
                Wf.reshape(1, D),
                jnp.broadcast_to(bf.reshape(1, 1), (1, D)))
    return y.reshape(-1)
